# Initial kernel scaffold; baseline (speedup 1.0000x reference)
import jax
import jax.numpy as jnp
from jax.experimental import pallas as pl


def kernel(x, pos, edge_index, params):
    raise NotImplementedError("write your pallas kernel here")


# speedup vs baseline: 1.2234x; 1.2234x over previous
import functools
import math

import jax
import jax.numpy as jnp
from jax.experimental import pallas as pl

N = 10000
E = 160000
C = 256
H = 8
HD = C // H
NB = 1000
EB = 1600
SCALE = 1.0 / math.sqrt(HD)


def _ln(y, g, b, eps=1e-5):
    m = jnp.mean(y, axis=-1, keepdims=True)
    v = jnp.mean((y - m) ** 2, axis=-1, keepdims=True)
    return (y - m) * jax.lax.rsqrt(v + eps) * g + b


def _proj_body(x_ref, w_ref, b_ref, q_ref, k_ref, v_ref):
    x = x_ref[...]
    y = jnp.dot(x, w_ref[...], preferred_element_type=jnp.float32) + b_ref[...]
    q_ref[...] = y[:, :C]
    k_ref[...] = y[:, C:2 * C]
    v_ref[...] = y[:, 2 * C:]


def _proj(x, wqkv, bqkv):
    grid = (N // NB,)
    return pl.pallas_call(
        _proj_body,
        grid=grid,
        in_specs=[
            pl.BlockSpec((NB, C), lambda i: (i, 0)),
            pl.BlockSpec((C, 3 * C), lambda i: (0, 0)),
            pl.BlockSpec((1, 3 * C), lambda i: (0, 0)),
        ],
        out_specs=[pl.BlockSpec((NB, C), lambda i: (i, 0))] * 3,
        out_shape=[jax.ShapeDtypeStruct((N, C), jnp.float32)] * 3,
    )(x, wqkv, bqkv)


def _edge_body(d_ref, qg_ref, kg_ref, vg_ref, w1_ref, b1_ref, w2_ref,
               b2_ref, g_ref, bln_ref, m_ref, w_ref, msg_ref):
    r = jnp.maximum(
        jnp.dot(d_ref[...], w1_ref[...], preferred_element_type=jnp.float32)
        + b1_ref[...], 0.0)
    pe = _ln(jnp.dot(r, w2_ref[...], preferred_element_type=jnp.float32)
             + b2_ref[...], g_ref[...], bln_ref[...])
    k = kg_ref[...] + pe
    mm = m_ref[...]
    s = jnp.dot(qg_ref[...] * k, mm, preferred_element_type=jnp.float32) * SCALE
    w = jnp.exp(s)
    w_ref[...] = w
    msg_ref[...] = jnp.dot(w, mm.T, preferred_element_type=jnp.float32) * vg_ref[...]


def _edge(dpad, qg, kg, vg, w1, b1, w2, b2, g, bln, mhead):
    grid = (E // EB,)
    return pl.pallas_call(
        _edge_body,
        grid=grid,
        in_specs=[
            pl.BlockSpec((EB, 8), lambda i: (i, 0)),
            pl.BlockSpec((EB, C), lambda i: (i, 0)),
            pl.BlockSpec((EB, C), lambda i: (i, 0)),
            pl.BlockSpec((EB, C), lambda i: (i, 0)),
            pl.BlockSpec((8, C), lambda i: (0, 0)),
            pl.BlockSpec((1, C), lambda i: (0, 0)),
            pl.BlockSpec((C, C), lambda i: (0, 0)),
            pl.BlockSpec((1, C), lambda i: (0, 0)),
            pl.BlockSpec((1, C), lambda i: (0, 0)),
            pl.BlockSpec((1, C), lambda i: (0, 0)),
            pl.BlockSpec((C, H), lambda i: (0, 0)),
        ],
        out_specs=[
            pl.BlockSpec((EB, H), lambda i: (i, 0)),
            pl.BlockSpec((EB, C), lambda i: (i, 0)),
        ],
        out_shape=[
            jax.ShapeDtypeStruct((E, H), jnp.float32),
            jax.ShapeDtypeStruct((E, C), jnp.float32),
        ],
    )(dpad, qg, kg, vg, w1, b1, w2, b2, g, bln, mhead)


def _post_body(num_ref, den_ref, xres_ref, mh_ref, wo_ref, bo_ref, g1_ref,
               bl1_ref, wf1_ref, bf1_ref, wf2_ref, bf2_ref, g2_ref, bl2_ref,
               out_ref):
    dfull = jnp.dot(den_ref[...], mh_ref[...].T,
                    preferred_element_type=jnp.float32)
    aggr = num_ref[...] / (dfull + 1e-16)
    out = jnp.dot(aggr, wo_ref[...], preferred_element_type=jnp.float32) + bo_ref[...]
    x1 = _ln(out + xres_ref[...], g1_ref[...], bl1_ref[...])
    h = jnp.maximum(
        jnp.dot(x1, wf1_ref[...], preferred_element_type=jnp.float32)
        + bf1_ref[...], 0.0)
    y = jnp.dot(h, wf2_ref[...], preferred_element_type=jnp.float32) + bf2_ref[...] + x1
    out_ref[...] = _ln(y, g2_ref[...], bl2_ref[...])


def _post(num, den, xres, mhead, wo, bo, g1, bl1, wf1, bf1, wf2, bf2, g2, bl2):
    grid = (N // NB,)
    return pl.pallas_call(
        _post_body,
        grid=grid,
        in_specs=[
            pl.BlockSpec((NB, C), lambda i: (i, 0)),
            pl.BlockSpec((NB, H), lambda i: (i, 0)),
            pl.BlockSpec((NB, C), lambda i: (i, 0)),
            pl.BlockSpec((C, H), lambda i: (0, 0)),
            pl.BlockSpec((C, C), lambda i: (0, 0)),
            pl.BlockSpec((1, C), lambda i: (0, 0)),
            pl.BlockSpec((1, C), lambda i: (0, 0)),
            pl.BlockSpec((1, C), lambda i: (0, 0)),
            pl.BlockSpec((C, 4 * C), lambda i: (0, 0)),
            pl.BlockSpec((1, 4 * C), lambda i: (0, 0)),
            pl.BlockSpec((4 * C, C), lambda i: (0, 0)),
            pl.BlockSpec((1, C), lambda i: (0, 0)),
            pl.BlockSpec((1, C), lambda i: (0, 0)),
            pl.BlockSpec((1, C), lambda i: (0, 0)),
        ],
        out_specs=pl.BlockSpec((NB, C), lambda i: (i, 0)),
        out_shape=jax.ShapeDtypeStruct((N, C), jnp.float32),
    )(num, den, xres, mhead, wo, bo, g1, bl1, wf1, bf1, wf2, bf2, g2, bl2)


def kernel(x, pos, edge_index, params):
    src = edge_index[0]
    dst = edge_index[1]
    mhead = jnp.repeat(jnp.eye(H, dtype=jnp.float32), HD, axis=0)
    d = pos[dst] - pos[src]
    dpad = jnp.pad(d, ((0, 0), (0, 5)))

    for p in params:
        wqkv = jnp.concatenate([p["q"]["W"], p["k"]["W"], p["v"]["W"]], axis=1)
        bqkv = jnp.concatenate([p["q"]["b"], p["k"]["b"], p["v"]["b"]])[None, :]
        q, k, v = _proj(x, wqkv, bqkv)
        qg = jnp.take(q, dst, axis=0)
        kg = jnp.take(k, src, axis=0)
        vg = jnp.take(v, src, axis=0)
        w1 = jnp.pad(p["pos1"]["W"], ((0, 5), (0, 0)))
        w, msg = _edge(dpad, qg, kg, vg, w1, p["pos1"]["b"][None, :],
                       p["pos2"]["W"], p["pos2"]["b"][None, :],
                       p["pos_ln"]["g"][None, :], p["pos_ln"]["b"][None, :],
                       mhead)
        num = jax.ops.segment_sum(msg, dst, N)
        den = jax.ops.segment_sum(w, dst, N)
        x = _post(num, den, x, mhead, p["out"]["W"], p["out"]["b"][None, :],
                  p["ln1"]["g"][None, :], p["ln1"]["b"][None, :],
                  p["ffn1"]["W"], p["ffn1"]["b"][None, :],
                  p["ffn2"]["W"], p["ffn2"]["b"][None, :],
                  p["ln2"]["g"][None, :], p["ln2"]["b"][None, :])
    return x
